# initial kernel scaffold (unmeasured)
import jax
import jax.numpy as jnp
from jax import lax
from jax.experimental import pallas as pl
from jax.experimental.pallas import tpu as pltpu


def kernel(
    x,
):
    def body(*refs):
        pass

    out_shape = jax.ShapeDtypeStruct(..., jnp.float32)
    return pl.pallas_call(body, out_shape=out_shape)(...)


# baseline (device time: 23580 ns/iter reference)
import jax
import jax.numpy as jnp
from jax import lax
from jax.experimental import pallas as pl
from jax.experimental.pallas import tpu as pltpu

N_DEV = 4
N_HOPS = 2 * (N_DEV - 1)


def kernel(x):
    m, n = x.shape
    mc = m // N_DEV

    def body(x_ref, out_ref, comm_ref, send_sems, recv_sems):
        my = lax.axis_index("i")
        left = (my - 1) % N_DEV
        right = (my + 1) % N_DEV

        barrier_sem = pltpu.get_barrier_semaphore()
        for nbr in [left, right]:
            pl.semaphore_signal(
                barrier_sem, inc=1,
                device_id=(nbr,), device_id_type=pl.DeviceIdType.MESH,
            )
        pl.semaphore_wait(barrier_sem, 2)

        comm_ref[0] = x_ref[pl.ds(my * mc, mc), :].astype(jnp.bfloat16)

        for h in range(N_HOPS):
            send_slot = h % 2
            recv_slot = (h + 1) % 2
            rdma = pltpu.make_async_remote_copy(
                src_ref=comm_ref.at[send_slot],
                dst_ref=comm_ref.at[recv_slot],
                send_sem=send_sems.at[h],
                recv_sem=recv_sems.at[h],
                device_id=(right,),
                device_id_type=pl.DeviceIdType.MESH,
            )
            rdma.start()
            rdma.wait()

            if h < N_DEV - 1:
                c = (my - h - 1) % N_DEV
                comm_ref[recv_slot] = (
                    comm_ref[recv_slot]
                    + x_ref[pl.ds(c * mc, mc), :].astype(jnp.bfloat16)
                )
                if h == N_DEV - 2:
                    rc = (my + 1) % N_DEV
                    out_ref[pl.ds(rc * mc, mc), :] = (
                        comm_ref[recv_slot].astype(jnp.float32)
                    )
            else:
                c = (my - (h - (N_DEV - 1))) % N_DEV
                out_ref[pl.ds(c * mc, mc), :] = (
                    comm_ref[recv_slot].astype(jnp.float32)
                )

    return pl.pallas_call(
        body,
        out_shape=jax.ShapeDtypeStruct((m, n), jnp.float32),
        in_specs=[pl.BlockSpec(memory_space=pltpu.VMEM)],
        out_specs=pl.BlockSpec(memory_space=pltpu.VMEM),
        scratch_shapes=[
            pltpu.VMEM((2, mc, n), jnp.bfloat16),
            pltpu.SemaphoreType.DMA((N_HOPS,)),
            pltpu.SemaphoreType.DMA((N_HOPS,)),
        ],
        compiler_params=pltpu.CompilerParams(collective_id=0),
    )(x)


# device time: 15033 ns/iter; 1.5685x vs baseline; 1.5685x over previous
import jax
import jax.numpy as jnp
from jax import lax
from jax.experimental import pallas as pl
from jax.experimental.pallas import tpu as pltpu

N_DEV = 4


def kernel(x):
    m, n = x.shape
    mc = m // N_DEV

    def body(
        x_ref,
        out_ref,
        rs_buf,
        ag_buf,
        rs_recv,
        ag_recv,
        rs_send_sems,
        rs_recv_sems,
        ag_send_sems,
        ag_recv_sems,
    ):
        my = lax.axis_index("i")

        barrier_sem = pltpu.get_barrier_semaphore()
        for o in (1, 2, 3):
            peer = (my + o) % N_DEV
            pl.semaphore_signal(
                barrier_sem, inc=1,
                device_id=(peer,), device_id_type=pl.DeviceIdType.MESH,
            )
        pl.semaphore_wait(barrier_sem, 3)


        rs_rdmas = []
        for o in (1, 2, 3):
            peer = (my + o) % N_DEV
            slot = 3 - o
            rs_buf[slot] = x_ref[pl.ds(peer * mc, mc), :].astype(jnp.bfloat16)
            r = pltpu.make_async_remote_copy(
                src_ref=rs_buf.at[slot],
                dst_ref=rs_recv.at[slot],
                send_sem=rs_send_sems.at[slot],
                recv_sem=rs_recv_sems.at[slot],
                device_id=(peer,),
                device_id_type=pl.DeviceIdType.MESH,
            )
            r.start()
            rs_rdmas.append(r)

        for r in rs_rdmas:
            r.wait_recv()

        acc = x_ref[pl.ds(my * mc, mc), :]
        acc = acc + rs_recv[0].astype(jnp.float32)
        acc = acc + rs_recv[1].astype(jnp.float32)
        acc = acc + rs_recv[2].astype(jnp.float32)
        out_ref[pl.ds(my * mc, mc), :] = acc
        ag_buf[...] = acc.astype(jnp.bfloat16)

        ag_rdmas = []
        for o in (1, 2, 3):
            peer = (my + o) % N_DEV
            slot = 3 - o
            r = pltpu.make_async_remote_copy(
                src_ref=ag_buf,
                dst_ref=ag_recv.at[slot],
                send_sem=ag_send_sems.at[slot],
                recv_sem=ag_recv_sems.at[slot],
                device_id=(peer,),
                device_id_type=pl.DeviceIdType.MESH,
            )
            r.start()
            ag_rdmas.append(r)

        for o, r in zip((1, 2, 3), ag_rdmas):
            slot = 3 - o
            r.wait_recv()
            origin = (my + slot + 1) % N_DEV
            out_ref[pl.ds(origin * mc, mc), :] = ag_recv[slot].astype(jnp.float32)

        for r in rs_rdmas:
            r.wait_send()
        for r in ag_rdmas:
            r.wait_send()

    return pl.pallas_call(
        body,
        out_shape=jax.ShapeDtypeStruct((m, n), jnp.float32),
        in_specs=[pl.BlockSpec(memory_space=pltpu.VMEM)],
        out_specs=pl.BlockSpec(memory_space=pltpu.VMEM),
        scratch_shapes=[
            pltpu.VMEM((N_DEV - 1, mc, n), jnp.bfloat16),
            pltpu.VMEM((mc, n), jnp.bfloat16),
            pltpu.VMEM((N_DEV - 1, mc, n), jnp.bfloat16),
            pltpu.VMEM((N_DEV - 1, mc, n), jnp.bfloat16),
            pltpu.SemaphoreType.DMA((N_DEV - 1,)),
            pltpu.SemaphoreType.DMA((N_DEV - 1,)),
            pltpu.SemaphoreType.DMA((N_DEV - 1,)),
            pltpu.SemaphoreType.DMA((N_DEV - 1,)),
        ],
        compiler_params=pltpu.CompilerParams(collective_id=0),
    )(x)
